# baseline (speedup 1.0000x reference)
import functools
import math

import jax
import jax.numpy as jnp
from jax import lax
from jax.experimental import pallas as pl
from jax.experimental.pallas import tpu as pltpu
from jax.experimental.pallas import tpu_sc as plsc

D_MODEL = 32
SCALE = float(math.sqrt(D_MODEL))
LANES = 16


@functools.lru_cache(maxsize=None)
def _build(N, S, D):
    NW = 32
    NB = N // NW
    ST = S // 8
    DT = D // 8
    TW = 8 * NB
    assert NB == 128 and S % 8 == 0 and D % 8 == 0

    mesh = plsc.VectorSubcoreMesh(core_axis_name="c", subcore_axis_name="s")

    @functools.partial(
        pl.kernel,
        mesh=mesh,
        out_type=jax.ShapeDtypeStruct((S, DT, NW, TW), jnp.float32),
        scratch_types=[
            pltpu.VMEM((S, NB), jnp.int32),
            pltpu.VMEM((2, NB, D), jnp.float32),
            pltpu.VMEM((2, DT * TW), jnp.float32),
            pltpu.SemaphoreType.DMA,
            pltpu.SemaphoreType.DMA,
            pltpu.SemaphoreType.DMA,
            pltpu.SemaphoreType.DMA,
            pltpu.SemaphoreType.DMA,
        ],
        compiler_params=pltpu.CompilerParams(
            use_tc_tiling_on_sc=False, needs_layout_passes=False
        ),
    )
    def emb(x_hbm, table_hbm, out_hbm, idx_v, rows_v, t_v,
            gsem0, gsem1, wsem0, wsem1, ssem):
        gsem = (gsem0, gsem1)
        wsem = (wsem0, wsem1)
        wid = lax.axis_index("s") * 2 + lax.axis_index("c")

        for st in range(ST):
            pltpu.async_copy(
                x_hbm.at[st, wid], idx_v.at[pl.ds(st * 8, 8)], ssem
            )
        for st in range(ST):
            pltpu.make_async_copy(
                x_hbm.at[0, 0], idx_v.at[pl.ds(st * 8, 8)], ssem
            ).wait()

        dvec = lax.iota(jnp.int32, LANES)
        doff0 = (dvec // 8) * TW + (dvec % 8) * NB
        doff1 = ((dvec + LANES) // 8) * TW + ((dvec + LANES) % 8) * NB

        def fire(s, b):
            pltpu.async_copy(table_hbm.at[idx_v.at[s]], rows_v.at[b], gsem[b])

        def drain(b):
            pltpu.make_async_copy(
                table_hbm.at[idx_v.at[0]], rows_v.at[b], gsem[b]
            ).wait()

        def writeback(s, b):
            for dt in range(DT):
                pltpu.async_copy(
                    t_v.at[b, pl.ds(dt * TW, TW)],
                    out_hbm.at[s, dt, wid],
                    wsem[b],
                )

        def drain_writeback(b):
            for dt in range(DT):
                pltpu.make_async_copy(
                    t_v.at[b, pl.ds(dt * TW, TW)],
                    out_hbm.at[0, dt, wid],
                    wsem[b],
                ).wait()

        def transpose_scale(b):
            tb = t_v.at[b]

            def body(i, c):
                for r in range(4):
                    n = i * 4 + r
                    ncol = jnp.full((LANES,), 0, jnp.int32) + n
                    v0 = rows_v[b, n, pl.ds(0, LANES)] * SCALE
                    v1 = rows_v[b, n, pl.ds(LANES, LANES)] * SCALE
                    plsc.store_scatter(tb, [doff0 + ncol], v0)
                    plsc.store_scatter(tb, [doff1 + ncol], v1)
                return c

            lax.fori_loop(0, NB // 4, body, 0)

        fire(0, 0)

        def body(h, carry):
            for b in range(2):
                s = 2 * h + b

                @pl.when(s + 1 < S)
                def _():
                    fire(s + 1, 1 - b)

                drain(b)

                @pl.when(s >= 2)
                def _():
                    drain_writeback(b)

                transpose_scale(b)
                writeback(s, b)
            return carry

        lax.fori_loop(0, S // 2, body, 0)
        drain_writeback(0)
        drain_writeback(1)

    return emb


def kernel(x, table):
    n, s = x.shape
    D = table.shape[1]
    x = x.astype(jnp.int32)
    NW = 32
    x4 = x.T.reshape(s // 8, 8, NW, n // NW).transpose(0, 2, 1, 3)
    out6 = _build(n, s, D)(x4, table)
    out = (
        out6.reshape(s, D // 8, NW, 8, n // NW)
        .transpose(2, 4, 0, 1, 3)
        .reshape(n, s, D)
    )
    return out


# speedup vs baseline: 1.0036x; 1.0036x over previous
import functools
import math

import jax
import jax.numpy as jnp
from jax import lax
from jax.experimental import pallas as pl
from jax.experimental.pallas import tpu as pltpu
from jax.experimental.pallas import tpu_sc as plsc

D_MODEL = 32
SCALE = float(math.sqrt(D_MODEL))
LANES = 16
NBUF = 4


@functools.lru_cache(maxsize=None)
def _build(N, S, D):
    NW = 32
    NB = N // NW
    ST = S // 8
    DT = D // 8
    TW = 8 * NB
    assert NB == 128 and S % 8 == 0 and D % 8 == 0 and S % NBUF == 0

    mesh = plsc.VectorSubcoreMesh(core_axis_name="c", subcore_axis_name="s")

    @functools.partial(
        pl.kernel,
        mesh=mesh,
        out_type=jax.ShapeDtypeStruct((S, DT, NW, TW), jnp.float32),
        scratch_types=[
            pltpu.VMEM((S, NB), jnp.int32),
            pltpu.VMEM((NBUF, NB, D), jnp.float32),
            pltpu.VMEM((NBUF, DT, TW), jnp.float32),
            [pltpu.SemaphoreType.DMA] * NBUF,
            [pltpu.SemaphoreType.DMA] * NBUF,
            pltpu.SemaphoreType.DMA,
        ],
        compiler_params=pltpu.CompilerParams(
            use_tc_tiling_on_sc=False, needs_layout_passes=False
        ),
    )
    def emb(x_hbm, table_hbm, out_hbm, idx_v, rows_v, t_v, gsem, wsem, ssem):
        wid = lax.axis_index("s") * 2 + lax.axis_index("c")

        for st in range(ST):
            pltpu.async_copy(
                x_hbm.at[st, wid], idx_v.at[pl.ds(st * 8, 8)], ssem
            )
        for st in range(ST):
            pltpu.make_async_copy(
                x_hbm.at[0, 0], idx_v.at[pl.ds(st * 8, 8)], ssem
            ).wait()

        dvec = lax.iota(jnp.int32, LANES)
        dsel = (dvec // 8, (dvec % 8) * NB)
        dvec1 = dvec + LANES
        dsel1 = (dvec1 // 8, (dvec1 % 8) * NB)

        def fire(s, b):
            pltpu.async_copy(table_hbm.at[idx_v.at[s]], rows_v.at[b], gsem[b])

        def drain(b):
            pltpu.make_async_copy(
                table_hbm.at[idx_v.at[0]], rows_v.at[b], gsem[b]
            ).wait()

        def fire_wb(s, b):
            pltpu.async_copy(t_v.at[b], out_hbm.at[s, :, wid], wsem[b])

        def drain_wb(b):
            pltpu.make_async_copy(
                t_v.at[b], out_hbm.at[0, :, wid], wsem[b]
            ).wait()

        def transpose_scale(b):
            tb = t_v.at[b]

            def body(i, c):
                for r in range(8):
                    n = i * 8 + r
                    ncol = jnp.full((LANES,), 0, jnp.int32) + n
                    v0 = rows_v[b, n, pl.ds(0, LANES)] * SCALE
                    v1 = rows_v[b, n, pl.ds(LANES, LANES)] * SCALE
                    plsc.store_scatter(tb, [dsel[0], dsel[1] + ncol], v0)
                    plsc.store_scatter(tb, [dsel1[0], dsel1[1] + ncol], v1)
                return c

            lax.fori_loop(0, NB // 8, body, 0)

        for b in range(NBUF - 1):
            fire(b, b)

        def body(h, carry):
            for b in range(NBUF):
                s = NBUF * h + b

                @pl.when(s + NBUF - 1 < S)
                def _():
                    fire(s + NBUF - 1, (b + NBUF - 1) % NBUF)

                drain(b)

                @pl.when(s >= NBUF)
                def _():
                    drain_wb(b)

                transpose_scale(b)
                fire_wb(s, b)
            return carry

        lax.fori_loop(0, S // NBUF, body, 0)
        for b in range(NBUF):
            drain_wb(b)

    return emb


def kernel(x, table):
    n, s = x.shape
    D = table.shape[1]
    x = x.astype(jnp.int32)
    NW = 32
    x4 = x.T.reshape(s // 8, 8, NW, n // NW).transpose(0, 2, 1, 3)
    out6 = _build(n, s, D)(x4, table)
    out = (
        out6.reshape(s, D // 8, NW, 8, n // NW)
        .transpose(2, 4, 0, 1, 3)
        .reshape(n, s, D)
    )
    return out


# speedup vs baseline: 1.5651x; 1.5594x over previous
import functools
import math

import jax
import jax.numpy as jnp
from jax import lax
from jax.experimental import pallas as pl
from jax.experimental.pallas import tpu as pltpu
from jax.experimental.pallas import tpu_sc as plsc

D_MODEL = 32
SCALE = float(math.sqrt(D_MODEL))
LANES = 16
NBUF = 4


@functools.lru_cache(maxsize=None)
def _build(N, S, D):
    NW = 32
    NB = N // NW
    ST = S // 8
    DT = D // 8
    TW = 8 * NB
    assert NB == 128 and S % 8 == 0 and D % 8 == 0 and S % NBUF == 0

    mesh = plsc.VectorSubcoreMesh(core_axis_name="c", subcore_axis_name="s")

    @functools.partial(
        pl.kernel,
        mesh=mesh,
        out_type=jax.ShapeDtypeStruct((S, DT, NW, 8, NB), jnp.float32),
        scratch_types=[
            pltpu.VMEM((S, NB), jnp.int32),
            pltpu.VMEM((NBUF, NB, D), jnp.float32),
            pltpu.VMEM((NBUF, DT, 8, NB + 1), jnp.float32),
            [pltpu.SemaphoreType.DMA] * NBUF,
            [pltpu.SemaphoreType.DMA] * NBUF,
            pltpu.SemaphoreType.DMA,
        ],
        compiler_params=pltpu.CompilerParams(
            use_tc_tiling_on_sc=False, needs_layout_passes=False
        ),
    )
    def emb(x_hbm, table_hbm, out_hbm, idx_v, rows_v, t_v, gsem, wsem, ssem):
        wid = lax.axis_index("s") * 2 + lax.axis_index("c")

        for st in range(ST):
            pltpu.async_copy(
                x_hbm.at[st, wid], idx_v.at[pl.ds(st * 8, 8)], ssem
            )
        for st in range(ST):
            pltpu.make_async_copy(
                x_hbm.at[0, 0], idx_v.at[pl.ds(st * 8, 8)], ssem
            ).wait()

        dvec = lax.iota(jnp.int32, LANES)
        dsel = (dvec // 8, dvec % 8)
        dvec1 = dvec + LANES
        dsel1 = (dvec1 // 8, dvec1 % 8)

        def fire(s, b):
            pltpu.async_copy(table_hbm.at[idx_v.at[s]], rows_v.at[b], gsem[b])

        def drain(b):
            pltpu.make_async_copy(
                table_hbm.at[idx_v.at[0]], rows_v.at[b], gsem[b]
            ).wait()

        def fire_wb(s, b):
            pltpu.async_copy(
                t_v.at[b, :, :, pl.ds(0, NB)], out_hbm.at[s, :, wid], wsem[b]
            )

        def drain_wb(b):
            pltpu.make_async_copy(
                t_v.at[b, :, :, pl.ds(0, NB)], out_hbm.at[0, :, wid], wsem[b]
            ).wait()

        def transpose_scale(b):
            tb = t_v.at[b]

            def body(i, c):
                for r in range(8):
                    n = i * 8 + r
                    ncol = jnp.full((LANES,), 0, jnp.int32) + n
                    v0 = rows_v[b, n, pl.ds(0, LANES)] * SCALE
                    v1 = rows_v[b, n, pl.ds(LANES, LANES)] * SCALE
                    plsc.store_scatter(tb, [dsel[0], dsel[1], ncol], v0)
                    plsc.store_scatter(tb, [dsel1[0], dsel1[1], ncol], v1)
                return c

            lax.fori_loop(0, NB // 8, body, 0)

        for b in range(NBUF - 1):
            fire(b, b)

        def body(h, carry):
            for b in range(NBUF):
                s = NBUF * h + b

                @pl.when(s + NBUF - 1 < S)
                def _():
                    fire(s + NBUF - 1, (b + NBUF - 1) % NBUF)

                drain(b)

                @pl.when(s >= NBUF)
                def _():
                    drain_wb(b)

                transpose_scale(b)
                fire_wb(s, b)
            return carry

        lax.fori_loop(0, S // NBUF, body, 0)
        for b in range(NBUF):
            drain_wb(b)

    return emb


def kernel(x, table):
    n, s = x.shape
    D = table.shape[1]
    x = x.astype(jnp.int32)
    NW = 32
    x4 = x.T.reshape(s // 8, 8, NW, n // NW).transpose(0, 2, 1, 3)
    out6 = _build(n, s, D)(x4, table)
    out = (
        out6.reshape(s, D // 8, NW, 8, n // NW)
        .transpose(2, 4, 0, 1, 3)
        .reshape(n, s, D)
    )
    return out


# speedup vs baseline: 2.4666x; 1.5760x over previous
import functools
import math

import jax
import jax.numpy as jnp
from jax import lax
from jax.experimental import pallas as pl
from jax.experimental.pallas import tpu as pltpu
from jax.experimental.pallas import tpu_sc as plsc

D_MODEL = 32
SCALE = float(math.sqrt(D_MODEL))
LANES = 16
NBUF = 4


@functools.lru_cache(maxsize=None)
def _build(N, S, D):
    NW = 32
    NB = N // NW
    ST = S // 8
    DT = D // 8
    TW = 8 * NB
    assert NB == 128 and S % 8 == 0 and D % 8 == 0 and S % NBUF == 0

    mesh = plsc.VectorSubcoreMesh(core_axis_name="c", subcore_axis_name="s")

    @functools.partial(
        pl.kernel,
        mesh=mesh,
        out_type=jax.ShapeDtypeStruct((S, DT, NW, 8, NB), jnp.float32),
        scratch_types=[
            pltpu.VMEM((S, NB), jnp.int32),
            pltpu.VMEM((NBUF, NB, D), jnp.float32),
            pltpu.VMEM((NBUF, DT, 8, NB + 1), jnp.float32),
            [pltpu.SemaphoreType.DMA] * NBUF,
            [pltpu.SemaphoreType.DMA] * NBUF,
            pltpu.SemaphoreType.DMA,
        ],
        compiler_params=pltpu.CompilerParams(
            use_tc_tiling_on_sc=False, needs_layout_passes=False
        ),
    )
    def emb(x_hbm, table_hbm, out_hbm, idx_v, rows_v, t_v, gsem, wsem, ssem):
        wid = lax.axis_index("s") * 2 + lax.axis_index("c")

        for st in range(ST):
            pltpu.async_copy(
                x_hbm.at[st, wid], idx_v.at[pl.ds(st * 8, 8)], ssem
            )
        for st in range(ST):
            pltpu.make_async_copy(
                x_hbm.at[0, 0], idx_v.at[pl.ds(st * 8, 8)], ssem
            ).wait()

        dvec = lax.iota(jnp.int32, LANES)
        dsel = (dvec // 8, dvec % 8)
        dvec1 = dvec + LANES
        dsel1 = (dvec1 // 8, dvec1 % 8)

        def fire(s, b):
            pltpu.async_copy(table_hbm.at[idx_v.at[s]], rows_v.at[b], gsem[b])

        def drain(b):
            pltpu.make_async_copy(
                table_hbm.at[idx_v.at[0]], rows_v.at[b], gsem[b]
            ).wait()

        def fire_wb(s, b):
            pltpu.async_copy(
                t_v.at[b, :, :, pl.ds(0, NB)], out_hbm.at[s, :, wid], wsem[b]
            )

        def drain_wb(b):
            pltpu.make_async_copy(
                t_v.at[b, :, :, pl.ds(0, NB)], out_hbm.at[0, :, wid], wsem[b]
            ).wait()

        def transpose_scale(b):
            tb = t_v.at[b]

            def body(i, c):
                for r in range(8):
                    n = i * 8 + r
                    ncol = jnp.full((LANES,), 0, jnp.int32) + n
                    v0 = rows_v[b, n, pl.ds(0, LANES)] * SCALE
                    v1 = rows_v[b, n, pl.ds(LANES, LANES)] * SCALE
                    plsc.store_scatter(tb, [dsel[0], dsel[1], ncol], v0)
                    plsc.store_scatter(tb, [dsel1[0], dsel1[1], ncol], v1)
                return c

            lax.fori_loop(0, NB // 8, body, 0)

        for b in range(NBUF - 1):
            fire(b, b)

        def body(h, carry):
            for b in range(NBUF):
                s = NBUF * h + b

                @pl.when(s + NBUF - 1 < S)
                def _():
                    fire(s + NBUF - 1, (b + NBUF - 1) % NBUF)

                drain(b)

                @pl.when(s >= NBUF)
                def _():
                    drain_wb(b)

                transpose_scale(b)
                fire_wb(s, b)
            return carry

        lax.fori_loop(0, S // NBUF, body, 0)
        for b in range(NBUF):
            drain_wb(b)

    return emb


_CB = 2048


def _tc_row_major(table):
    V, D = table.shape
    t32 = table.T
    G = pl.cdiv(V, 4 * _CB)

    def body(t_ref, o_ref):
        blk = t_ref[...]
        o_ref[...] = jnp.concatenate(
            [blk[:, r * _CB : (r + 1) * _CB].T for r in range(4)], axis=1
        )

    out = pl.pallas_call(
        body,
        grid=(G,),
        in_specs=[pl.BlockSpec((D, 4 * _CB), lambda g: (0, g))],
        out_specs=pl.BlockSpec((_CB, 4 * D), lambda g: (g, 0)),
        out_shape=jax.ShapeDtypeStruct((G * _CB, 4 * D), jnp.float32),
    )(t32)
    return out.reshape(4 * G * _CB, D)


def _permute_idx(x):
    lg = _CB.bit_length() - 1
    return (x & ~(4 * _CB - 1)) | ((x & (_CB - 1)) << 2) | ((x >> lg) & 3)


def kernel(x, table):
    n, s = x.shape
    D = table.shape[1]
    x = _permute_idx(x.astype(jnp.int32))
    NW = 32
    x4 = x.T.reshape(s // 8, 8, NW, n // NW).transpose(0, 2, 1, 3)
    out6 = _build(n, s, D)(x4, _tc_row_major(table))
    out = (
        out6.reshape(s, D // 8, NW, 8, n // NW)
        .transpose(2, 4, 0, 1, 3)
        .reshape(n, s, D)
    )
    return out


# speedup vs baseline: 3.3702x; 1.3664x over previous
import functools
import math

import jax
import jax.numpy as jnp
from jax import lax
from jax.experimental import pallas as pl
from jax.experimental.pallas import tpu as pltpu
from jax.experimental.pallas import tpu_sc as plsc

D_MODEL = 32
SCALE = float(math.sqrt(D_MODEL))
LANES = 16
NBUF = 4


@functools.lru_cache(maxsize=None)
def _build(N, S, D):
    NW = 32
    NB = N // NW
    ST = S // 8
    DT = D // 8
    TW = 8 * NB
    assert NB == 128 and S % 8 == 0 and D % 8 == 0 and S % NBUF == 0

    mesh = plsc.VectorSubcoreMesh(core_axis_name="c", subcore_axis_name="s")

    @functools.partial(
        pl.kernel,
        mesh=mesh,
        out_type=jax.ShapeDtypeStruct((S, DT, NW, 8, NB), jnp.float32),
        scratch_types=[
            pltpu.VMEM((S, NB), jnp.int32),
            pltpu.VMEM((NBUF, NB, D), jnp.float32),
            pltpu.VMEM((NBUF, DT, 8, NB + 1), jnp.float32),
            [pltpu.SemaphoreType.DMA] * NBUF,
            [pltpu.SemaphoreType.DMA] * NBUF,
            pltpu.SemaphoreType.DMA,
        ],
        compiler_params=pltpu.CompilerParams(
            use_tc_tiling_on_sc=False, needs_layout_passes=False
        ),
    )
    def emb(x_hbm, table_hbm, out_hbm, idx_v, rows_v, t_v, gsem, wsem, ssem):
        wid = lax.axis_index("s") * 2 + lax.axis_index("c")

        for st in range(ST):
            pltpu.async_copy(
                x_hbm.at[st, wid], idx_v.at[pl.ds(st * 8, 8)], ssem
            )
        for st in range(ST):
            pltpu.make_async_copy(
                x_hbm.at[0, 0], idx_v.at[pl.ds(st * 8, 8)], ssem
            ).wait()

        dvec = lax.iota(jnp.int32, LANES)
        dsel = (dvec // 8, dvec % 8)
        dvec1 = dvec + LANES
        dsel1 = (dvec1 // 8, dvec1 % 8)

        def fire(s, b):
            pltpu.async_copy(table_hbm.at[idx_v.at[s]], rows_v.at[b], gsem[b])

        def drain(b):
            pltpu.make_async_copy(
                table_hbm.at[idx_v.at[0]], rows_v.at[b], gsem[b]
            ).wait()

        def fire_wb(s, b):
            pltpu.async_copy(
                t_v.at[b, :, :, pl.ds(0, NB)], out_hbm.at[s, :, wid], wsem[b]
            )

        def drain_wb(b):
            pltpu.make_async_copy(
                t_v.at[b, :, :, pl.ds(0, NB)], out_hbm.at[0, :, wid], wsem[b]
            ).wait()

        def transpose_scale(b):
            tb = t_v.at[b]

            def body(i, c):
                for r in range(8):
                    n = i * 8 + r
                    ncol = jnp.full((LANES,), 0, jnp.int32) + n
                    v0 = rows_v[b, n, pl.ds(0, LANES)] * SCALE
                    v1 = rows_v[b, n, pl.ds(LANES, LANES)] * SCALE
                    plsc.store_scatter(tb, [dsel[0], dsel[1], ncol], v0)
                    plsc.store_scatter(tb, [dsel1[0], dsel1[1], ncol], v1)
                return c

            lax.fori_loop(0, NB // 8, body, 0)

        for b in range(NBUF - 1):
            fire(b, b)

        def body(h, carry):
            for b in range(NBUF):
                s = NBUF * h + b

                @pl.when(s + NBUF - 1 < S)
                def _():
                    fire(s + NBUF - 1, (b + NBUF - 1) % NBUF)

                drain(b)

                @pl.when(s >= NBUF)
                def _():
                    drain_wb(b)

                transpose_scale(b)
                fire_wb(s, b)
            return carry

        lax.fori_loop(0, S // NBUF, body, 0)
        for b in range(NBUF):
            drain_wb(b)

    return emb


_CB = 2048


def _tc_row_major(table):
    V, D = table.shape
    t32 = table.T
    G = pl.cdiv(V, 4 * _CB)

    def body(t_ref, o_ref):
        blk = t_ref[...]
        o_ref[...] = jnp.concatenate(
            [blk[:, r * _CB : (r + 1) * _CB] for r in range(4)], axis=0
        ).T

    out = pl.pallas_call(
        body,
        grid=(G,),
        in_specs=[pl.BlockSpec((D, 4 * _CB), lambda g: (0, g))],
        out_specs=pl.BlockSpec((_CB, 4 * D), lambda g: (g, 0)),
        out_shape=jax.ShapeDtypeStruct((G * _CB, 4 * D), jnp.float32),
    )(t32)
    return out.reshape(4 * G * _CB, D)


def _permute_idx(x):
    lg = _CB.bit_length() - 1
    return (x & ~(4 * _CB - 1)) | ((x & (_CB - 1)) << 2) | ((x >> lg) & 3)


def kernel(x, table):
    n, s = x.shape
    D = table.shape[1]
    x = _permute_idx(x.astype(jnp.int32))
    NW = 32
    x4 = x.T.reshape(s // 8, 8, NW, n // NW).transpose(0, 2, 1, 3)
    out6 = _build(n, s, D)(x4, _tc_row_major(table))
    out = (
        out6.reshape(s, D // 8, NW, 8, n // NW)
        .transpose(2, 4, 0, 1, 3)
        .reshape(n, s, D)
    )
    return out


# speedup vs baseline: 3.5684x; 1.0588x over previous
import functools
import math

import jax
import jax.numpy as jnp
from jax import lax
from jax.experimental import pallas as pl
from jax.experimental.pallas import tpu as pltpu
from jax.experimental.pallas import tpu_sc as plsc

D_MODEL = 32
SCALE = float(math.sqrt(D_MODEL))
LANES = 16
NBUF = 8


@functools.lru_cache(maxsize=None)
def _build(N, S, D):
    NW = 32
    NB = N // NW
    ST = S // 8
    DT = D // 8
    TW = 8 * NB
    assert NB == 128 and S % 8 == 0 and D % 8 == 0 and S % NBUF == 0

    mesh = plsc.VectorSubcoreMesh(core_axis_name="c", subcore_axis_name="s")

    @functools.partial(
        pl.kernel,
        mesh=mesh,
        out_type=jax.ShapeDtypeStruct((S, DT, NW, 8, NB), jnp.float32),
        scratch_types=[
            pltpu.VMEM((S, NB), jnp.int32),
            pltpu.VMEM((NBUF, NB, D), jnp.float32),
            pltpu.VMEM((NBUF, DT, 8, NB + 1), jnp.float32),
            [pltpu.SemaphoreType.DMA] * NBUF,
            [pltpu.SemaphoreType.DMA] * NBUF,
            pltpu.SemaphoreType.DMA,
        ],
        compiler_params=pltpu.CompilerParams(
            use_tc_tiling_on_sc=False, needs_layout_passes=False
        ),
    )
    def emb(x_hbm, table_hbm, out_hbm, idx_v, rows_v, t_v, gsem, wsem, ssem):
        wid = lax.axis_index("s") * 2 + lax.axis_index("c")

        for st in range(ST):
            pltpu.async_copy(
                x_hbm.at[st, wid], idx_v.at[pl.ds(st * 8, 8)], ssem
            )
        for st in range(ST):
            pltpu.make_async_copy(
                x_hbm.at[0, 0], idx_v.at[pl.ds(st * 8, 8)], ssem
            ).wait()

        dvec = lax.iota(jnp.int32, LANES)
        dsel = (dvec // 8, dvec % 8)
        dvec1 = dvec + LANES
        dsel1 = (dvec1 // 8, dvec1 % 8)

        def fire(s, b):
            pltpu.async_copy(table_hbm.at[idx_v.at[s]], rows_v.at[b], gsem[b])

        def drain(b):
            pltpu.make_async_copy(
                table_hbm.at[idx_v.at[0]], rows_v.at[b], gsem[b]
            ).wait()

        def fire_wb(s, b):
            pltpu.async_copy(
                t_v.at[b, :, :, pl.ds(0, NB)], out_hbm.at[s, :, wid], wsem[b]
            )

        def drain_wb(b):
            pltpu.make_async_copy(
                t_v.at[b, :, :, pl.ds(0, NB)], out_hbm.at[0, :, wid], wsem[b]
            ).wait()

        def transpose_scale(b):
            tb = t_v.at[b]

            def body(i, c):
                for r in range(8):
                    n = i * 8 + r
                    ncol = jnp.full((LANES,), 0, jnp.int32) + n
                    v0 = rows_v[b, n, pl.ds(0, LANES)] * SCALE
                    v1 = rows_v[b, n, pl.ds(LANES, LANES)] * SCALE
                    plsc.store_scatter(tb, [dsel[0], dsel[1], ncol], v0)
                    plsc.store_scatter(tb, [dsel1[0], dsel1[1], ncol], v1)
                return c

            lax.fori_loop(0, NB // 8, body, 0)

        for b in range(NBUF - 1):
            fire(b, b)

        def body(h, carry):
            for b in range(NBUF):
                s = NBUF * h + b

                @pl.when(s + NBUF - 1 < S)
                def _():
                    fire(s + NBUF - 1, (b + NBUF - 1) % NBUF)

                drain(b)

                @pl.when(s >= NBUF)
                def _():
                    drain_wb(b)

                transpose_scale(b)
                fire_wb(s, b)
            return carry

        lax.fori_loop(0, S // NBUF, body, 0)
        for b in range(NBUF):
            drain_wb(b)

    return emb


_CB = 4096


def _tc_row_major(table):
    V, D = table.shape
    t32 = table.T
    G = pl.cdiv(V, 4 * _CB)

    def body(t_ref, o_ref):
        blk = t_ref[...]
        o_ref[...] = jnp.concatenate(
            [blk[:, r * _CB : (r + 1) * _CB] for r in range(4)], axis=0
        ).T

    out = pl.pallas_call(
        body,
        grid=(G,),
        in_specs=[pl.BlockSpec((D, 4 * _CB), lambda g: (0, g))],
        out_specs=pl.BlockSpec((_CB, 4 * D), lambda g: (g, 0)),
        out_shape=jax.ShapeDtypeStruct((G * _CB, 4 * D), jnp.float32),
    )(t32)
    return out.reshape(4 * G * _CB, D)


def _permute_idx(x):
    lg = _CB.bit_length() - 1
    return (x & ~(4 * _CB - 1)) | ((x & (_CB - 1)) << 2) | ((x >> lg) & 3)


def kernel(x, table):
    n, s = x.shape
    D = table.shape[1]
    x = _permute_idx(x.astype(jnp.int32))
    NW = 32
    x4 = x.T.reshape(s // 8, 8, NW, n // NW).transpose(0, 2, 1, 3)
    out6 = _build(n, s, D)(x4, _tc_row_major(table))
    out = (
        out6.reshape(s, D // 8, NW, 8, n // NW)
        .transpose(2, 4, 0, 1, 3)
        .reshape(n, s, D)
    )
    return out
